# baseline (speedup 1.0000x reference)
import functools

import jax
import jax.numpy as jnp
from jax import lax
from jax.experimental import pallas as pl
from jax.experimental.pallas import tpu as pltpu
from jax.experimental.pallas import tpu_sc as plsc

N = 10000
NR = 10240
E = 320000
E_PAD = 327680
G = 64
CHUNK = 128
ROWS_PER_SUB = NR // 16
STRIPES = ROWS_PER_SUB // CHUNK


def _zero_rows(buf, nrows, width):
    def body(i, _):
        for j in range(width // 16):
            buf[i, pl.ds(j * 16, 16)] = jnp.zeros((16,), jnp.float32)
        return 0
    lax.fori_loop(0, nrows, body, 0)


def _deg_kernel(dst2d, degp, dst_v, ones_v, zeros_v, acc):
    c = lax.axis_index("c")
    s = lax.axis_index("s")
    n_chunks = E_PAD // (32 * CHUNK)
    base = (c * 16 + s) * n_chunks

    def fill_ones(i, _):
        ones_v[i, pl.ds(0, 16)] = jnp.ones((16,), jnp.float32)
        return 0
    lax.fori_loop(0, CHUNK, fill_ones, 0)
    _zero_rows(zeros_v, CHUNK, 16)

    for j in range(STRIPES):
        r = s * ROWS_PER_SUB + j * CHUNK
        pltpu.sync_copy(zeros_v, acc.at[pl.ds(r, CHUNK)])
    pltpu.sync_copy(dst2d.at[pl.ds(base, n_chunks)], dst_v)
    plsc.subcore_barrier()

    def body(i, _):
        pltpu.sync_copy(ones_v, acc.at[dst_v.at[i]], add=True)
        return 0
    lax.fori_loop(0, n_chunks, body, 0)
    plsc.subcore_barrier()
    for j in range(STRIPES):
        r = s * ROWS_PER_SUB + j * CHUNK
        pltpu.sync_copy(acc.at[pl.ds(r, CHUNK)],
                        degp.at[pl.ds(c * NR + r, CHUNK)])


GRP = 32
CK = 128
NBUF = 2


def _layer_kernel(half, hwp, src2d, dst2d, outp,
                  src_v, dst_v, rows, acc, sgs, sss):
    c = lax.axis_index("c")
    s = lax.axis_index("s")
    n_chunks = E_PAD // (16 * CK)
    n_groups = n_chunks // GRP
    base = s * n_chunks

    _zero_rows(rows[0], CK, half)
    for j in range(STRIPES):
        r = s * ROWS_PER_SUB + j * CHUNK
        pltpu.sync_copy(rows[0], acc.at[pl.ds(r, CHUNK)])
    plsc.subcore_barrier()

    off = c * NR
    L = NBUF - 1

    def group(g, _):
        gb = base + g * GRP
        pltpu.sync_copy(src2d.at[pl.ds(gb, GRP)], src_v)
        pltpu.sync_copy(dst2d.at[pl.ds(gb, GRP)], dst_v)
        for i in range(GRP):
            for j in range(CK // 16):
                sl = pl.ds(j * 16, 16)
                src_v[i, sl] = src_v[i, sl] + off
        gathers = [None] * NBUF
        scatters = [None] * NBUF
        for k in range(L):
            gathers[k] = pltpu.async_copy(
                hwp.at[src_v.at[k]], rows[k], sgs[k])
        for i in range(GRP):
            b = i % NBUF
            pb = (i - 1) % NBUF
            if scatters[pb] is not None:
                scatters[pb].wait()
                scatters[pb] = None
            if i + L < GRP:
                b2 = (i + L) % NBUF
                gathers[b2] = pltpu.async_copy(
                    hwp.at[src_v.at[i + L]], rows[b2], sgs[b2])
            gathers[b].wait()
            scatters[b] = pltpu.async_copy(
                rows[b], acc.at[dst_v.at[i]], sss[b], add=True)
        for k in range(NBUF):
            if scatters[k] is not None:
                scatters[k].wait()
        return 0
    lax.fori_loop(0, n_groups, group, 0)
    plsc.subcore_barrier()
    for j in range(STRIPES):
        r = s * ROWS_PER_SUB + j * CHUNK
        pltpu.sync_copy(acc.at[pl.ds(r, CHUNK)],
                        outp.at[pl.ds(c * NR + r, CHUNK)])


def _layer_kernel_es(fo, hw, src2d, dst2d, outp,
                     src_v, dst_v, rows, acc, sgs, sss):
    c = lax.axis_index("c")
    s = lax.axis_index("s")
    n_chunks = E_PAD // (32 * CK)
    n_groups = n_chunks // 16
    base = (c * 16 + s) * n_chunks

    _zero_rows(rows[0], CK, fo)
    for j in range(STRIPES):
        r = s * ROWS_PER_SUB + j * CHUNK
        pltpu.sync_copy(rows[0], acc.at[pl.ds(r, CHUNK)])
    plsc.subcore_barrier()

    def group(g, _):
        gb = base + g * 16
        pltpu.sync_copy(src2d.at[pl.ds(gb, 16)], src_v)
        pltpu.sync_copy(dst2d.at[pl.ds(gb, 16)], dst_v)
        gathers = [None, None]
        scatters = [None, None]
        gathers[0] = pltpu.async_copy(hw.at[src_v.at[0]], rows[0], sgs[0])
        for i in range(16):
            b = i % 2
            nb = 1 - b
            if i + 1 < 16:
                if scatters[nb] is not None:
                    scatters[nb].wait()
                gathers[nb] = pltpu.async_copy(
                    hw.at[src_v.at[i + 1]], rows[nb], sgs[nb])
            gathers[b].wait()
            scatters[b] = pltpu.async_copy(
                rows[b], acc.at[dst_v.at[i]], sss[b], add=True)
        scatters[0].wait()
        scatters[1].wait()
        return 0
    lax.fori_loop(0, n_groups, group, 0)
    plsc.subcore_barrier()
    for j in range(STRIPES):
        r = s * ROWS_PER_SUB + j * CHUNK
        pltpu.sync_copy(acc.at[pl.ds(r, CHUNK)],
                        outp.at[pl.ds(c * NR + r, CHUNK)])


def _make_layer_es_call(fo):
    mesh = plsc.VectorSubcoreMesh(core_axis_name="c", subcore_axis_name="s")
    return pl.kernel(
        functools.partial(_layer_kernel_es, fo),
        out_type=jax.ShapeDtypeStruct((2 * NR, fo), jnp.float32),
        mesh=mesh,
        compiler_params=pltpu.CompilerParams(use_tc_tiling_on_sc=False),
        scratch_types=[
            pltpu.VMEM((16, CK), jnp.int32),
            pltpu.VMEM((16, CK), jnp.int32),
            tuple(pltpu.VMEM((CK, fo), jnp.float32) for _ in range(2)),
            pltpu.VMEM_SHARED((NR, fo), jnp.float32),
            tuple(pltpu.SemaphoreType.DMA for _ in range(2)),
            tuple(pltpu.SemaphoreType.DMA for _ in range(2)),
        ],
    )


def _make_deg_call():
    mesh = plsc.VectorSubcoreMesh(core_axis_name="c", subcore_axis_name="s")
    return pl.kernel(
        _deg_kernel,
        out_type=jax.ShapeDtypeStruct((2 * NR, 16), jnp.float32),
        mesh=mesh,
        scratch_types=[
            pltpu.VMEM((E_PAD // (32 * CHUNK), CHUNK), jnp.int32),
            pltpu.VMEM((CHUNK, 16), jnp.float32),
            pltpu.VMEM((CHUNK, 16), jnp.float32),
            pltpu.VMEM_SHARED((NR, 16), jnp.float32),
        ],
    )


def _make_layer_call(half):
    mesh = plsc.VectorSubcoreMesh(core_axis_name="c", subcore_axis_name="s")
    return pl.kernel(
        functools.partial(_layer_kernel, half),
        out_type=jax.ShapeDtypeStruct((2 * NR, half), jnp.float32),
        mesh=mesh,
        compiler_params=pltpu.CompilerParams(use_tc_tiling_on_sc=False),
        scratch_types=[
            pltpu.VMEM((GRP, CK), jnp.int32),
            pltpu.VMEM((GRP, CK), jnp.int32),
            tuple(pltpu.VMEM((CK, half), jnp.float32) for _ in range(NBUF)),
            pltpu.VMEM_SHARED((NR, half), jnp.float32),
            tuple(pltpu.SemaphoreType.DMA for _ in range(NBUF)),
            tuple(pltpu.SemaphoreType.DMA for _ in range(NBUF)),
        ],
    )


def _tc1_body(x_ref, w_ref, degp_ref, hw_ref, dinv_ref):
    deg = degp_ref[0:NR, 0:1] + degp_ref[NR:2 * NR, 0:1] + 1.0
    dinv = lax.rsqrt(deg)
    dinv_ref[...] = dinv
    hw_ref[...] = jnp.dot(x_ref[...], w_ref[...],
                          preferred_element_type=jnp.float32) * dinv


def _tc_mid_body(pack_out, sp_ref, hw_ref, dinv_ref, b_ref, w_ref, out_ref):
    dinv = dinv_ref[...]
    ssum = sp_ref[0:NR, :] + sp_ref[NR:2 * NR, :]
    h = jnp.maximum(dinv * (ssum + hw_ref[...]) + b_ref[0:1, :], 0.0)
    t = jnp.dot(h, w_ref[...], preferred_element_type=jnp.float32) * dinv
    if pack_out:
        oh = t.shape[1] // 2
        out_ref[0:NR, :] = t[:, 0:oh]
        out_ref[NR:2 * NR, :] = t[:, oh:]
    else:
        out_ref[...] = t


def _tc_h3_body(sp_ref, hwp_ref, dinv_ref, b_ref, h3_ref):
    dinv = dinv_ref[...]
    half = 128
    hA = jnp.maximum(dinv * (sp_ref[0:NR, :] + hwp_ref[0:NR, :])
                     + b_ref[0:1, 0:half], 0.0)
    hB = jnp.maximum(dinv * (sp_ref[NR:2 * NR, :] + hwp_ref[NR:2 * NR, :])
                     + b_ref[0:1, half:], 0.0)
    h3_ref[:, 0:half] = hA
    h3_ref[:, half:] = hB


def _tc_gate_body(h3_ref, gw1_ref, gb1_ref, gw2_ref, gb2_ref, gw3_ref,
                  gb3_ref, gate_ref):
    g = jnp.maximum(jnp.dot(h3_ref[...], gw1_ref[...],
                            preferred_element_type=jnp.float32)
                    + gb1_ref[0:1, :], 0.0)
    g = jnp.maximum(jnp.dot(g, gw2_ref[...],
                            preferred_element_type=jnp.float32)
                    + gb2_ref[0:1, :], 0.0)
    gate_ref[...] = jnp.dot(g, gw3_ref[...],
                            preferred_element_type=jnp.float32) + gb3_ref[0:1, :]


def _tc_pool_body(h3_ref, gate_ref, batch_ref, hw1_ref, hb1_ref, hw2_ref,
                  hb2_ref, hw3_ref, hb3_ref, o_ref):
    batch = batch_ref[...]
    seg = lax.broadcasted_iota(jnp.int32, (NR, G), 1)
    m = (batch == seg).astype(jnp.float32)
    valid = (batch < G)
    gate = gate_ref[...]
    gb = jnp.where(m > 0.0, gate, -1e30)
    smax = jnp.max(gb, axis=0, keepdims=True)
    smax_node = jnp.sum(m * smax, axis=1, keepdims=True)
    e = jnp.where(valid, jnp.exp(gate - smax_node), 0.0)
    den = jnp.sum(m * e, axis=0, keepdims=True)
    den_node = jnp.sum(m * den, axis=1, keepdims=True)
    alpha = e / jnp.where(den_node > 0.0, den_node, 1.0)
    ah = alpha * h3_ref[...]
    pooled = lax.dot_general(m, ah, (((0,), (0,)), ((), ())),
                             preferred_element_type=jnp.float32)
    o = jnp.maximum(jnp.dot(pooled, hw1_ref[...],
                            preferred_element_type=jnp.float32)
                    + hb1_ref[0:1, :], 0.0)
    o = jnp.maximum(jnp.dot(o, hw2_ref[...],
                            preferred_element_type=jnp.float32)
                    + hb2_ref[0:1, :], 0.0)
    o_ref[...] = jnp.dot(o, hw3_ref[...],
                         preferred_element_type=jnp.float32) + hb3_ref[0:1, :]


def _tc_call(body, out_shapes):
    return pl.pallas_call(body, out_shape=out_shapes)


def kernel(x, edge_index, batch, W1, b1, W2, b2, W3, b3, gW1, gb1, gW2, gb2,
           gW3, gb3, hW1, hb1, hW2, hb2, hW3, hb3):
    x_pad = jnp.zeros((NR, 128), jnp.float32).at[:N, :].set(x)
    src2d = jnp.full((E_PAD,), N, jnp.int32).at[:E].set(edge_index[0])
    src2d = src2d.reshape(E_PAD // CK, CK)
    dst2d = jnp.full((E_PAD,), N, jnp.int32).at[:E].set(edge_index[1])
    dst2d = dst2d.reshape(E_PAD // CK, CK)
    dst2d_deg = dst2d.reshape(E_PAD // CHUNK, CHUNK)
    batch_pad = jnp.full((NR,), G, jnp.int32).at[:N].set(batch)
    batch_pad = batch_pad.reshape(NR, 1)
    b1r = b1.reshape(1, -1)
    b2r = b2.reshape(1, -1)
    b3r = b3.reshape(1, -1)
    gb1r, gb2r, gb3r = gb1.reshape(1, -1), gb2.reshape(1, -1), gb3.reshape(1, -1)
    hb1r, hb2r, hb3r = hb1.reshape(1, -1), hb2.reshape(1, -1), hb3.reshape(1, -1)

    degp = _make_deg_call()(dst2d_deg)

    hw1, dinv = _tc_call(
        _tc1_body,
        [jax.ShapeDtypeStruct((NR, 64), jnp.float32),
         jax.ShapeDtypeStruct((NR, 1), jnp.float32)])(x_pad, W1, degp)

    sp1 = _make_layer_es_call(64)(hw1, src2d, dst2d)
    hw2 = _tc_call(
        functools.partial(_tc_mid_body, False),
        jax.ShapeDtypeStruct((NR, 128), jnp.float32))(
            sp1, hw1, dinv, b1r, W2)

    sp2 = _make_layer_es_call(128)(hw2, src2d, dst2d)
    hwp3 = _tc_call(
        functools.partial(_tc_mid_body, True),
        jax.ShapeDtypeStruct((2 * NR, 128), jnp.float32))(
            sp2, hw2, dinv, b2r, W3)

    sp3 = _make_layer_call(128)(hwp3, src2d, dst2d)
    h3 = _tc_call(
        _tc_h3_body,
        jax.ShapeDtypeStruct((NR, 256), jnp.float32))(sp3, hwp3, dinv, b3r)

    gate = _tc_call(
        _tc_gate_body,
        jax.ShapeDtypeStruct((NR, 1), jnp.float32))(
            h3, gW1, gb1r, gW2, gb2r, gW3, gb3r)
    o = _tc_call(
        _tc_pool_body,
        jax.ShapeDtypeStruct((G, 1), jnp.float32))(
            h3, gate, batch_pad, hW1, hb1r, hW2, hb2r, hW3, hb3r)
    return o.reshape(-1)


# speedup vs baseline: 1.2116x; 1.2116x over previous
import functools

import jax
import jax.numpy as jnp
from jax import lax
from jax.experimental import pallas as pl
from jax.experimental.pallas import tpu as pltpu
from jax.experimental.pallas import tpu_sc as plsc

N = 10000
NR = 10240
E = 320000
E_PAD = 327680
G = 64
CHUNK = 128
ROWS_PER_SUB = NR // 16
STRIPES = ROWS_PER_SUB // CHUNK


def _zero_rows(buf, nrows, width):
    def body(i, _):
        for j in range(width // 16):
            buf[i, pl.ds(j * 16, 16)] = jnp.zeros((16,), jnp.float32)
        return 0
    lax.fori_loop(0, nrows, body, 0)


def _deg_kernel(dst2d, degp, dst_v, ones_v, zeros_v, acc):
    c = lax.axis_index("c")
    s = lax.axis_index("s")
    n_chunks = E_PAD // (32 * CHUNK)
    base = (c * 16 + s) * n_chunks

    def fill_ones(i, _):
        ones_v[i, pl.ds(0, 16)] = jnp.ones((16,), jnp.float32)
        return 0
    lax.fori_loop(0, CHUNK, fill_ones, 0)
    _zero_rows(zeros_v, CHUNK, 16)

    for j in range(STRIPES):
        r = s * ROWS_PER_SUB + j * CHUNK
        pltpu.sync_copy(zeros_v, acc.at[pl.ds(r, CHUNK)])
    pltpu.sync_copy(dst2d.at[pl.ds(base, n_chunks)], dst_v)
    plsc.subcore_barrier()

    def body(i, _):
        pltpu.sync_copy(ones_v, acc.at[dst_v.at[i]], add=True)
        return 0
    lax.fori_loop(0, n_chunks, body, 0)
    plsc.subcore_barrier()
    for j in range(STRIPES):
        r = s * ROWS_PER_SUB + j * CHUNK
        pltpu.sync_copy(acc.at[pl.ds(r, CHUNK)],
                        degp.at[pl.ds(c * NR + r, CHUNK)])


CK = 128


def _layer_kernel_es(fo, hw, src2d, dst2d, outp,
                     src_v, dst_v, rows, acc, sgs, sss):
    c = lax.axis_index("c")
    s = lax.axis_index("s")
    n_chunks = E_PAD // (32 * CK)
    n_groups = n_chunks // 16
    base = (c * 16 + s) * n_chunks

    _zero_rows(rows[0], CK, fo)
    for j in range(STRIPES):
        r = s * ROWS_PER_SUB + j * CHUNK
        pltpu.sync_copy(rows[0], acc.at[pl.ds(r, CHUNK)])
    plsc.subcore_barrier()

    def group(g, _):
        gb = base + g * 16
        pltpu.sync_copy(src2d.at[pl.ds(gb, 16)], src_v)
        pltpu.sync_copy(dst2d.at[pl.ds(gb, 16)], dst_v)
        gathers = [None, None]
        scatters = [None, None]
        gathers[0] = pltpu.async_copy(hw.at[src_v.at[0]], rows[0], sgs[0])
        for i in range(16):
            b = i % 2
            nb = 1 - b
            if i + 1 < 16:
                if scatters[nb] is not None:
                    scatters[nb].wait()
                gathers[nb] = pltpu.async_copy(
                    hw.at[src_v.at[i + 1]], rows[nb], sgs[nb])
            gathers[b].wait()
            scatters[b] = pltpu.async_copy(
                rows[b], acc.at[dst_v.at[i]], sss[b], add=True)
        scatters[0].wait()
        scatters[1].wait()
        return 0
    lax.fori_loop(0, n_groups, group, 0)
    plsc.subcore_barrier()
    for j in range(STRIPES):
        r = s * ROWS_PER_SUB + j * CHUNK
        pltpu.sync_copy(acc.at[pl.ds(r, CHUNK)],
                        outp.at[pl.ds(c * NR + r, CHUNK)])


def _make_layer_es_call(fo):
    mesh = plsc.VectorSubcoreMesh(core_axis_name="c", subcore_axis_name="s")
    return pl.kernel(
        functools.partial(_layer_kernel_es, fo),
        out_type=jax.ShapeDtypeStruct((2 * NR, fo), jnp.float32),
        mesh=mesh,
        compiler_params=pltpu.CompilerParams(use_tc_tiling_on_sc=False),
        scratch_types=[
            pltpu.VMEM((16, CK), jnp.int32),
            pltpu.VMEM((16, CK), jnp.int32),
            tuple(pltpu.VMEM((CK, fo), jnp.float32) for _ in range(2)),
            pltpu.VMEM_SHARED((NR, fo), jnp.float32),
            tuple(pltpu.SemaphoreType.DMA for _ in range(2)),
            tuple(pltpu.SemaphoreType.DMA for _ in range(2)),
        ],
    )


def _make_deg_call():
    mesh = plsc.VectorSubcoreMesh(core_axis_name="c", subcore_axis_name="s")
    return pl.kernel(
        _deg_kernel,
        out_type=jax.ShapeDtypeStruct((2 * NR, 16), jnp.float32),
        mesh=mesh,
        scratch_types=[
            pltpu.VMEM((E_PAD // (32 * CHUNK), CHUNK), jnp.int32),
            pltpu.VMEM((CHUNK, 16), jnp.float32),
            pltpu.VMEM((CHUNK, 16), jnp.float32),
            pltpu.VMEM_SHARED((NR, 16), jnp.float32),
        ],
    )


def _tc1_body(x_ref, w_ref, degp_ref, hw_ref, dinv_ref):
    deg = degp_ref[0:NR, 0:1] + degp_ref[NR:2 * NR, 0:1] + 1.0
    dinv = lax.rsqrt(deg)
    dinv_ref[...] = dinv
    hw_ref[...] = jnp.dot(x_ref[...], w_ref[...],
                          preferred_element_type=jnp.float32) * dinv


def _tc_u2_body(sp_ref, hw_ref, dinv_ref, b_ref, u2_ref):
    dinv = dinv_ref[...]
    ssum = sp_ref[0:NR, :] + sp_ref[NR:2 * NR, :]
    h = jnp.maximum(dinv * (ssum + hw_ref[...]) + b_ref[0:1, :], 0.0)
    u2_ref[...] = h * dinv


def _tc_u3_body(sp_ref, u2_ref, dinv_ref, b_ref, w_ref, u3_ref):
    dinv = dinv_ref[...]
    ssum = sp_ref[0:NR, :] + sp_ref[NR:2 * NR, :] + u2_ref[...]
    t = jnp.dot(ssum, w_ref[...], preferred_element_type=jnp.float32)
    u3_ref[...] = jnp.maximum(dinv * t + b_ref[0:1, :], 0.0) * dinv


def _tc_h3_body(sp_ref, u3_ref, dinv_ref, b_ref, w_ref, h3_ref):
    dinv = dinv_ref[...]
    ssum = sp_ref[0:NR, :] + sp_ref[NR:2 * NR, :] + u3_ref[...]
    t = jnp.dot(ssum, w_ref[...], preferred_element_type=jnp.float32)
    h3_ref[...] = jnp.maximum(dinv * t + b_ref[0:1, :], 0.0)


def _tc_gate_body(h3_ref, gw1_ref, gb1_ref, gw2_ref, gb2_ref, gw3_ref,
                  gb3_ref, gate_ref):
    g = jnp.maximum(jnp.dot(h3_ref[...], gw1_ref[...],
                            preferred_element_type=jnp.float32)
                    + gb1_ref[0:1, :], 0.0)
    g = jnp.maximum(jnp.dot(g, gw2_ref[...],
                            preferred_element_type=jnp.float32)
                    + gb2_ref[0:1, :], 0.0)
    gate_ref[...] = jnp.dot(g, gw3_ref[...],
                            preferred_element_type=jnp.float32) + gb3_ref[0:1, :]


def _tc_pool_body(h3_ref, gate_ref, batch_ref, hw1_ref, hb1_ref, hw2_ref,
                  hb2_ref, hw3_ref, hb3_ref, o_ref):
    batch = batch_ref[...]
    seg = lax.broadcasted_iota(jnp.int32, (NR, G), 1)
    m = (batch == seg).astype(jnp.float32)
    valid = (batch < G)
    gate = gate_ref[...]
    gb = jnp.where(m > 0.0, gate, -1e30)
    smax = jnp.max(gb, axis=0, keepdims=True)
    smax_node = jnp.sum(m * smax, axis=1, keepdims=True)
    e = jnp.where(valid, jnp.exp(gate - smax_node), 0.0)
    den = jnp.sum(m * e, axis=0, keepdims=True)
    den_node = jnp.sum(m * den, axis=1, keepdims=True)
    alpha = e / jnp.where(den_node > 0.0, den_node, 1.0)
    ah = alpha * h3_ref[...]
    pooled = lax.dot_general(m, ah, (((0,), (0,)), ((), ())),
                             preferred_element_type=jnp.float32)
    o = jnp.maximum(jnp.dot(pooled, hw1_ref[...],
                            preferred_element_type=jnp.float32)
                    + hb1_ref[0:1, :], 0.0)
    o = jnp.maximum(jnp.dot(o, hw2_ref[...],
                            preferred_element_type=jnp.float32)
                    + hb2_ref[0:1, :], 0.0)
    o_ref[...] = jnp.dot(o, hw3_ref[...],
                         preferred_element_type=jnp.float32) + hb3_ref[0:1, :]


def _tc_call(body, out_shapes):
    return pl.pallas_call(body, out_shape=out_shapes)


def kernel(x, edge_index, batch, W1, b1, W2, b2, W3, b3, gW1, gb1, gW2, gb2,
           gW3, gb3, hW1, hb1, hW2, hb2, hW3, hb3):
    x_pad = jnp.zeros((NR, 128), jnp.float32).at[:N, :].set(x)
    src2d = jnp.full((E_PAD,), N, jnp.int32).at[:E].set(edge_index[0])
    src2d = src2d.reshape(E_PAD // CK, CK)
    dst2d = jnp.full((E_PAD,), N, jnp.int32).at[:E].set(edge_index[1])
    dst2d = dst2d.reshape(E_PAD // CK, CK)
    dst2d_deg = dst2d.reshape(E_PAD // CHUNK, CHUNK)
    batch_pad = jnp.full((NR,), G, jnp.int32).at[:N].set(batch)
    batch_pad = batch_pad.reshape(NR, 1)
    b1r = b1.reshape(1, -1)
    b2r = b2.reshape(1, -1)
    b3r = b3.reshape(1, -1)
    gb1r, gb2r, gb3r = gb1.reshape(1, -1), gb2.reshape(1, -1), gb3.reshape(1, -1)
    hb1r, hb2r, hb3r = hb1.reshape(1, -1), hb2.reshape(1, -1), hb3.reshape(1, -1)

    degp = _make_deg_call()(dst2d_deg)

    hw1, dinv = _tc_call(
        _tc1_body,
        [jax.ShapeDtypeStruct((NR, 64), jnp.float32),
         jax.ShapeDtypeStruct((NR, 1), jnp.float32)])(x_pad, W1, degp)

    sp1 = _make_layer_es_call(64)(hw1, src2d, dst2d)
    u2 = _tc_call(
        _tc_u2_body,
        jax.ShapeDtypeStruct((NR, 64), jnp.float32))(sp1, hw1, dinv, b1r)

    sp2 = _make_layer_es_call(64)(u2, src2d, dst2d)
    u3 = _tc_call(
        _tc_u3_body,
        jax.ShapeDtypeStruct((NR, 128), jnp.float32))(
            sp2, u2, dinv, b2r, W2)

    sp3 = _make_layer_es_call(128)(u3, src2d, dst2d)
    h3 = _tc_call(
        _tc_h3_body,
        jax.ShapeDtypeStruct((NR, 256), jnp.float32))(
            sp3, u3, dinv, b3r, W3)

    gate = _tc_call(
        _tc_gate_body,
        jax.ShapeDtypeStruct((NR, 1), jnp.float32))(
            h3, gW1, gb1r, gW2, gb2r, gW3, gb3r)
    o = _tc_call(
        _tc_pool_body,
        jax.ShapeDtypeStruct((G, 1), jnp.float32))(
            h3, gate, batch_pad, hW1, hb1r, hW2, hb2r, hW3, hb3r)
    return o.reshape(-1)


# speedup vs baseline: 1.2263x; 1.0121x over previous
import functools

import jax
import jax.numpy as jnp
from jax import lax
from jax.experimental import pallas as pl
from jax.experimental.pallas import tpu as pltpu
from jax.experimental.pallas import tpu_sc as plsc

N = 10000
NR = 10240
E = 320000
E_PAD = 327680
G = 64
CHUNK = 128
ROWS_PER_SUB = NR // 16
STRIPES = ROWS_PER_SUB // CHUNK


def _zero_rows(buf, nrows, width):
    def body(i, _):
        for j in range(width // 16):
            buf[i, pl.ds(j * 16, 16)] = jnp.zeros((16,), jnp.float32)
        return 0
    lax.fori_loop(0, nrows, body, 0)


def _deg_kernel(dst2d, degp, dst_v, ones_v, zeros_v, acc):
    c = lax.axis_index("c")
    s = lax.axis_index("s")
    n_chunks = E_PAD // (32 * CHUNK)
    base = (c * 16 + s) * n_chunks

    def fill_ones(i, _):
        ones_v[i, pl.ds(0, 16)] = jnp.ones((16,), jnp.float32)
        return 0
    lax.fori_loop(0, CHUNK, fill_ones, 0)
    _zero_rows(zeros_v, CHUNK, 16)

    for j in range(STRIPES):
        r = s * ROWS_PER_SUB + j * CHUNK
        pltpu.sync_copy(zeros_v, acc.at[pl.ds(r, CHUNK)])
    pltpu.sync_copy(dst2d.at[pl.ds(base, n_chunks)], dst_v)
    plsc.subcore_barrier()

    def body(i, _):
        pltpu.sync_copy(ones_v, acc.at[dst_v.at[i]], add=True)
        return 0
    lax.fori_loop(0, n_chunks, body, 0)
    plsc.subcore_barrier()
    for j in range(STRIPES):
        r = s * ROWS_PER_SUB + j * CHUNK
        pltpu.sync_copy(acc.at[pl.ds(r, CHUNK)],
                        degp.at[pl.ds(c * NR + r, CHUNK)])


CK = 128


def _layer_kernel_es(fo, hw, src2d, dst2d, outp,
                     src_v, dst_v, rows, acc, sgs, sss):
    c = lax.axis_index("c")
    s = lax.axis_index("s")
    n_chunks = E_PAD // (32 * CK)
    n_groups = n_chunks // 16
    base = (c * 16 + s) * n_chunks

    _zero_rows(rows[0], CK, fo)
    for j in range(STRIPES):
        r = s * ROWS_PER_SUB + j * CHUNK
        pltpu.sync_copy(rows[0], acc.at[pl.ds(r, CHUNK)])
    plsc.subcore_barrier()

    def group(g, _):
        gb = base + g * 16
        pltpu.sync_copy(src2d.at[pl.ds(gb, 16)], src_v)
        pltpu.sync_copy(dst2d.at[pl.ds(gb, 16)], dst_v)
        gathers = [None, None]
        scatters = [None, None]
        gathers[0] = pltpu.async_copy(hw.at[src_v.at[0]], rows[0], sgs[0])
        for i in range(16):
            b = i % 2
            nb = 1 - b
            if i + 1 < 16:
                if scatters[nb] is not None:
                    scatters[nb].wait()
                gathers[nb] = pltpu.async_copy(
                    hw.at[src_v.at[i + 1]], rows[nb], sgs[nb])
            gathers[b].wait()
            scatters[b] = pltpu.async_copy(
                rows[b], acc.at[dst_v.at[i]], sss[b], add=True)
        scatters[0].wait()
        scatters[1].wait()
        return 0
    lax.fori_loop(0, n_groups, group, 0)
    plsc.subcore_barrier()
    for j in range(STRIPES):
        r = s * ROWS_PER_SUB + j * CHUNK
        pltpu.sync_copy(acc.at[pl.ds(r, CHUNK)],
                        outp.at[pl.ds(c * NR + r, CHUNK)])


NBUF2 = 3


def _layer_kernel_es2(fo, hw, src2d, dst2d, outp,
                      src_v, dst_v, rows, acc, sgs, sss):
    c = lax.axis_index("c")
    s = lax.axis_index("s")
    n_chunks = E_PAD // (32 * CK)
    base = (c * 16 + s) * n_chunks

    _zero_rows(rows[0], CK, fo)
    for j in range(STRIPES):
        r = s * ROWS_PER_SUB + j * CHUNK
        pltpu.sync_copy(rows[0], acc.at[pl.ds(r, CHUNK)])
    plsc.subcore_barrier()

    pltpu.sync_copy(src2d.at[pl.ds(base, n_chunks)], src_v)
    pltpu.sync_copy(dst2d.at[pl.ds(base, n_chunks)], dst_v)

    L = NBUF2 - 1
    gathers = [None] * NBUF2
    scatters = [None] * NBUF2
    for k in range(L):
        gathers[k] = pltpu.async_copy(hw.at[src_v.at[k]], rows[k], sgs[k])
    for i in range(n_chunks):
        b = i % NBUF2
        if i + L < n_chunks:
            b2 = (i + L) % NBUF2
            if scatters[b2] is not None:
                scatters[b2].wait()
                scatters[b2] = None
            gathers[b2] = pltpu.async_copy(
                hw.at[src_v.at[i + L]], rows[b2], sgs[b2])
        gathers[b].wait()
        scatters[b] = pltpu.async_copy(
            rows[b], acc.at[dst_v.at[i]], sss[b], add=True)
    for k in range(NBUF2):
        if scatters[k] is not None:
            scatters[k].wait()
    plsc.subcore_barrier()
    for j in range(STRIPES):
        r = s * ROWS_PER_SUB + j * CHUNK
        pltpu.sync_copy(acc.at[pl.ds(r, CHUNK)],
                        outp.at[pl.ds(c * NR + r, CHUNK)])


def _make_layer_es2_call(fo):
    mesh = plsc.VectorSubcoreMesh(core_axis_name="c", subcore_axis_name="s")
    n_chunks = E_PAD // (32 * CK)
    return pl.kernel(
        functools.partial(_layer_kernel_es2, fo),
        out_type=jax.ShapeDtypeStruct((2 * NR, fo), jnp.float32),
        mesh=mesh,
        compiler_params=pltpu.CompilerParams(use_tc_tiling_on_sc=False),
        scratch_types=[
            pltpu.VMEM((n_chunks, CK), jnp.int32),
            pltpu.VMEM((n_chunks, CK), jnp.int32),
            tuple(pltpu.VMEM((CK, fo), jnp.float32) for _ in range(NBUF2)),
            pltpu.VMEM_SHARED((NR, fo), jnp.float32),
            tuple(pltpu.SemaphoreType.DMA for _ in range(NBUF2)),
            tuple(pltpu.SemaphoreType.DMA for _ in range(NBUF2)),
        ],
    )


def _make_layer_es_call(fo):
    mesh = plsc.VectorSubcoreMesh(core_axis_name="c", subcore_axis_name="s")
    return pl.kernel(
        functools.partial(_layer_kernel_es, fo),
        out_type=jax.ShapeDtypeStruct((2 * NR, fo), jnp.float32),
        mesh=mesh,
        compiler_params=pltpu.CompilerParams(use_tc_tiling_on_sc=False),
        scratch_types=[
            pltpu.VMEM((16, CK), jnp.int32),
            pltpu.VMEM((16, CK), jnp.int32),
            tuple(pltpu.VMEM((CK, fo), jnp.float32) for _ in range(2)),
            pltpu.VMEM_SHARED((NR, fo), jnp.float32),
            tuple(pltpu.SemaphoreType.DMA for _ in range(2)),
            tuple(pltpu.SemaphoreType.DMA for _ in range(2)),
        ],
    )


def _make_deg_call():
    mesh = plsc.VectorSubcoreMesh(core_axis_name="c", subcore_axis_name="s")
    return pl.kernel(
        _deg_kernel,
        out_type=jax.ShapeDtypeStruct((2 * NR, 16), jnp.float32),
        mesh=mesh,
        scratch_types=[
            pltpu.VMEM((E_PAD // (32 * CHUNK), CHUNK), jnp.int32),
            pltpu.VMEM((CHUNK, 16), jnp.float32),
            pltpu.VMEM((CHUNK, 16), jnp.float32),
            pltpu.VMEM_SHARED((NR, 16), jnp.float32),
        ],
    )


def _tc1_body(x_ref, w_ref, degp_ref, hw_ref, dinv_ref):
    deg = degp_ref[0:NR, 0:1] + degp_ref[NR:2 * NR, 0:1] + 1.0
    dinv = lax.rsqrt(deg)
    dinv_ref[...] = dinv
    hw_ref[...] = jnp.dot(x_ref[...], w_ref[...],
                          preferred_element_type=jnp.float32) * dinv


def _tc_u2_body(sp_ref, hw_ref, dinv_ref, b_ref, u2_ref):
    dinv = dinv_ref[...]
    ssum = sp_ref[0:NR, :] + sp_ref[NR:2 * NR, :]
    h = jnp.maximum(dinv * (ssum + hw_ref[...]) + b_ref[0:1, :], 0.0)
    u2_ref[...] = h * dinv


def _tc_u3_body(sp_ref, u2_ref, dinv_ref, b_ref, w_ref, u3_ref):
    dinv = dinv_ref[...]
    ssum = sp_ref[0:NR, :] + sp_ref[NR:2 * NR, :] + u2_ref[...]
    t = jnp.dot(ssum, w_ref[...], preferred_element_type=jnp.float32)
    u3_ref[...] = jnp.maximum(dinv * t + b_ref[0:1, :], 0.0) * dinv


def _tc_h3_body(sp_ref, u3_ref, dinv_ref, b_ref, w_ref, h3_ref):
    dinv = dinv_ref[...]
    ssum = sp_ref[0:NR, :] + sp_ref[NR:2 * NR, :] + u3_ref[...]
    t = jnp.dot(ssum, w_ref[...], preferred_element_type=jnp.float32)
    h3_ref[...] = jnp.maximum(dinv * t + b_ref[0:1, :], 0.0)


def _tc_gate_body(h3_ref, gw1_ref, gb1_ref, gw2_ref, gb2_ref, gw3_ref,
                  gb3_ref, gate_ref):
    g = jnp.maximum(jnp.dot(h3_ref[...], gw1_ref[...],
                            preferred_element_type=jnp.float32)
                    + gb1_ref[0:1, :], 0.0)
    g = jnp.maximum(jnp.dot(g, gw2_ref[...],
                            preferred_element_type=jnp.float32)
                    + gb2_ref[0:1, :], 0.0)
    gate_ref[...] = jnp.dot(g, gw3_ref[...],
                            preferred_element_type=jnp.float32) + gb3_ref[0:1, :]


def _tc_pool_body(h3_ref, gate_ref, batch_ref, hw1_ref, hb1_ref, hw2_ref,
                  hb2_ref, hw3_ref, hb3_ref, o_ref):
    batch = batch_ref[...]
    seg = lax.broadcasted_iota(jnp.int32, (NR, G), 1)
    m = (batch == seg).astype(jnp.float32)
    valid = (batch < G)
    gate = gate_ref[...]
    gb = jnp.where(m > 0.0, gate, -1e30)
    smax = jnp.max(gb, axis=0, keepdims=True)
    smax_node = jnp.sum(m * smax, axis=1, keepdims=True)
    e = jnp.where(valid, jnp.exp(gate - smax_node), 0.0)
    den = jnp.sum(m * e, axis=0, keepdims=True)
    den_node = jnp.sum(m * den, axis=1, keepdims=True)
    alpha = e / jnp.where(den_node > 0.0, den_node, 1.0)
    ah = alpha * h3_ref[...]
    pooled = lax.dot_general(m, ah, (((0,), (0,)), ((), ())),
                             preferred_element_type=jnp.float32)
    o = jnp.maximum(jnp.dot(pooled, hw1_ref[...],
                            preferred_element_type=jnp.float32)
                    + hb1_ref[0:1, :], 0.0)
    o = jnp.maximum(jnp.dot(o, hw2_ref[...],
                            preferred_element_type=jnp.float32)
                    + hb2_ref[0:1, :], 0.0)
    o_ref[...] = jnp.dot(o, hw3_ref[...],
                         preferred_element_type=jnp.float32) + hb3_ref[0:1, :]


def _tc_call(body, out_shapes):
    return pl.pallas_call(body, out_shape=out_shapes)


def kernel(x, edge_index, batch, W1, b1, W2, b2, W3, b3, gW1, gb1, gW2, gb2,
           gW3, gb3, hW1, hb1, hW2, hb2, hW3, hb3):
    x_pad = jnp.zeros((NR, 128), jnp.float32).at[:N, :].set(x)
    src2d = jnp.full((E_PAD,), N, jnp.int32).at[:E].set(edge_index[0])
    src2d = src2d.reshape(E_PAD // CK, CK)
    dst2d = jnp.full((E_PAD,), N, jnp.int32).at[:E].set(edge_index[1])
    dst2d = dst2d.reshape(E_PAD // CK, CK)
    dst2d_deg = dst2d.reshape(E_PAD // CHUNK, CHUNK)
    batch_pad = jnp.full((NR,), G, jnp.int32).at[:N].set(batch)
    batch_pad = batch_pad.reshape(NR, 1)
    b1r = b1.reshape(1, -1)
    b2r = b2.reshape(1, -1)
    b3r = b3.reshape(1, -1)
    gb1r, gb2r, gb3r = gb1.reshape(1, -1), gb2.reshape(1, -1), gb3.reshape(1, -1)
    hb1r, hb2r, hb3r = hb1.reshape(1, -1), hb2.reshape(1, -1), hb3.reshape(1, -1)

    degp = _make_deg_call()(dst2d_deg)

    hw1, dinv = _tc_call(
        _tc1_body,
        [jax.ShapeDtypeStruct((NR, 64), jnp.float32),
         jax.ShapeDtypeStruct((NR, 1), jnp.float32)])(x_pad, W1, degp)

    sp1 = _make_layer_es2_call(64)(hw1, src2d, dst2d)
    u2 = _tc_call(
        _tc_u2_body,
        jax.ShapeDtypeStruct((NR, 64), jnp.float32))(sp1, hw1, dinv, b1r)

    sp2 = _make_layer_es2_call(64)(u2, src2d, dst2d)
    u3 = _tc_call(
        _tc_u3_body,
        jax.ShapeDtypeStruct((NR, 128), jnp.float32))(
            sp2, u2, dinv, b2r, W2)

    sp3 = _make_layer_es_call(128)(u3, src2d, dst2d)
    h3 = _tc_call(
        _tc_h3_body,
        jax.ShapeDtypeStruct((NR, 256), jnp.float32))(
            sp3, u3, dinv, b3r, W3)

    gate = _tc_call(
        _tc_gate_body,
        jax.ShapeDtypeStruct((NR, 1), jnp.float32))(
            h3, gW1, gb1r, gW2, gb2r, gW3, gb3r)
    o = _tc_call(
        _tc_pool_body,
        jax.ShapeDtypeStruct((G, 1), jnp.float32))(
            h3, gate, batch_pad, hW1, hb1r, hW2, hb2r, hW3, hb3r)
    return o.reshape(-1)


# speedup vs baseline: 1.2321x; 1.0047x over previous
import functools

import jax
import jax.numpy as jnp
from jax import lax
from jax.experimental import pallas as pl
from jax.experimental.pallas import tpu as pltpu
from jax.experimental.pallas import tpu_sc as plsc

N = 10000
NR = 10240
E = 320000
E_PAD = 327680
G = 64
CHUNK = 128
ROWS_PER_SUB = NR // 16
STRIPES = ROWS_PER_SUB // CHUNK


def _zero_rows(buf, nrows, width):
    def body(i, _):
        for j in range(width // 16):
            buf[i, pl.ds(j * 16, 16)] = jnp.zeros((16,), jnp.float32)
        return 0
    lax.fori_loop(0, nrows, body, 0)


def _deg_kernel(dst2d, degp, dst_v, ones_v, zeros_v, acc):
    c = lax.axis_index("c")
    s = lax.axis_index("s")
    n_chunks = E_PAD // (32 * CHUNK)
    base = (c * 16 + s) * n_chunks

    def fill_ones(i, _):
        ones_v[i, pl.ds(0, 16)] = jnp.ones((16,), jnp.float32)
        return 0
    lax.fori_loop(0, CHUNK, fill_ones, 0)
    _zero_rows(zeros_v, CHUNK, 16)

    for j in range(STRIPES):
        r = s * ROWS_PER_SUB + j * CHUNK
        pltpu.sync_copy(zeros_v, acc.at[pl.ds(r, CHUNK)])
    pltpu.sync_copy(dst2d.at[pl.ds(base, n_chunks)], dst_v)
    plsc.subcore_barrier()

    def body(i, _):
        pltpu.sync_copy(ones_v, acc.at[dst_v.at[i]], add=True)
        return 0
    lax.fori_loop(0, n_chunks, body, 0)
    plsc.subcore_barrier()
    for j in range(STRIPES):
        r = s * ROWS_PER_SUB + j * CHUNK
        pltpu.sync_copy(acc.at[pl.ds(r, CHUNK)],
                        degp.at[pl.ds(c * NR + r, CHUNK)])


CK = 128


GRP_ES = 40


def _layer_kernel_es(fo, hw, src2d, dst2d, outp,
                     src_v, dst_v, rows, acc, sgs, sss):
    c = lax.axis_index("c")
    s = lax.axis_index("s")
    n_chunks = E_PAD // (32 * CK)
    n_groups = n_chunks // GRP_ES
    base = (c * 16 + s) * n_chunks

    _zero_rows(rows[0], CK, fo)
    for j in range(STRIPES):
        r = s * ROWS_PER_SUB + j * CHUNK
        pltpu.sync_copy(rows[0], acc.at[pl.ds(r, CHUNK)])
    plsc.subcore_barrier()

    def group(g, _):
        gb = base + g * GRP_ES
        pltpu.sync_copy(src2d.at[pl.ds(gb, GRP_ES)], src_v)
        pltpu.sync_copy(dst2d.at[pl.ds(gb, GRP_ES)], dst_v)
        gathers = [None, None]
        scatters = [None, None]
        gathers[0] = pltpu.async_copy(hw.at[src_v.at[0]], rows[0], sgs[0])
        for i in range(GRP_ES):
            b = i % 2
            nb = 1 - b
            if i + 1 < GRP_ES:
                if scatters[nb] is not None:
                    scatters[nb].wait()
                gathers[nb] = pltpu.async_copy(
                    hw.at[src_v.at[i + 1]], rows[nb], sgs[nb])
            gathers[b].wait()
            scatters[b] = pltpu.async_copy(
                rows[b], acc.at[dst_v.at[i]], sss[b], add=True)
        scatters[0].wait()
        scatters[1].wait()
        return 0
    lax.fori_loop(0, n_groups, group, 0)
    plsc.subcore_barrier()
    for j in range(STRIPES):
        r = s * ROWS_PER_SUB + j * CHUNK
        pltpu.sync_copy(acc.at[pl.ds(r, CHUNK)],
                        outp.at[pl.ds(c * NR + r, CHUNK)])


NBUF2 = 3


def _layer_kernel_es2(fo, hw, src2d, dst2d, outp,
                      src_v, dst_v, rows, acc, sgs, sss):
    c = lax.axis_index("c")
    s = lax.axis_index("s")
    n_chunks = E_PAD // (32 * CK)
    base = (c * 16 + s) * n_chunks

    _zero_rows(rows[0], CK, fo)
    for j in range(STRIPES):
        r = s * ROWS_PER_SUB + j * CHUNK
        pltpu.sync_copy(rows[0], acc.at[pl.ds(r, CHUNK)])
    plsc.subcore_barrier()

    pltpu.sync_copy(src2d.at[pl.ds(base, n_chunks)], src_v)
    pltpu.sync_copy(dst2d.at[pl.ds(base, n_chunks)], dst_v)

    L = NBUF2 - 1
    gathers = [None] * NBUF2
    scatters = [None] * NBUF2
    for k in range(L):
        gathers[k] = pltpu.async_copy(hw.at[src_v.at[k]], rows[k], sgs[k])
    for i in range(n_chunks):
        b = i % NBUF2
        if i + L < n_chunks:
            b2 = (i + L) % NBUF2
            if scatters[b2] is not None:
                scatters[b2].wait()
                scatters[b2] = None
            gathers[b2] = pltpu.async_copy(
                hw.at[src_v.at[i + L]], rows[b2], sgs[b2])
        gathers[b].wait()
        scatters[b] = pltpu.async_copy(
            rows[b], acc.at[dst_v.at[i]], sss[b], add=True)
    for k in range(NBUF2):
        if scatters[k] is not None:
            scatters[k].wait()
    plsc.subcore_barrier()
    for j in range(STRIPES):
        r = s * ROWS_PER_SUB + j * CHUNK
        pltpu.sync_copy(acc.at[pl.ds(r, CHUNK)],
                        outp.at[pl.ds(c * NR + r, CHUNK)])


def _make_layer_es2_call(fo):
    mesh = plsc.VectorSubcoreMesh(core_axis_name="c", subcore_axis_name="s")
    n_chunks = E_PAD // (32 * CK)
    return pl.kernel(
        functools.partial(_layer_kernel_es2, fo),
        out_type=jax.ShapeDtypeStruct((2 * NR, fo), jnp.float32),
        mesh=mesh,
        compiler_params=pltpu.CompilerParams(use_tc_tiling_on_sc=False),
        scratch_types=[
            pltpu.VMEM((n_chunks, CK), jnp.int32),
            pltpu.VMEM((n_chunks, CK), jnp.int32),
            tuple(pltpu.VMEM((CK, fo), jnp.float32) for _ in range(NBUF2)),
            pltpu.VMEM_SHARED((NR, fo), jnp.float32),
            tuple(pltpu.SemaphoreType.DMA for _ in range(NBUF2)),
            tuple(pltpu.SemaphoreType.DMA for _ in range(NBUF2)),
        ],
    )


def _make_layer_es_call(fo):
    mesh = plsc.VectorSubcoreMesh(core_axis_name="c", subcore_axis_name="s")
    return pl.kernel(
        functools.partial(_layer_kernel_es, fo),
        out_type=jax.ShapeDtypeStruct((2 * NR, fo), jnp.float32),
        mesh=mesh,
        compiler_params=pltpu.CompilerParams(use_tc_tiling_on_sc=False),
        scratch_types=[
            pltpu.VMEM((GRP_ES, CK), jnp.int32),
            pltpu.VMEM((GRP_ES, CK), jnp.int32),
            tuple(pltpu.VMEM((CK, fo), jnp.float32) for _ in range(2)),
            pltpu.VMEM_SHARED((NR, fo), jnp.float32),
            tuple(pltpu.SemaphoreType.DMA for _ in range(2)),
            tuple(pltpu.SemaphoreType.DMA for _ in range(2)),
        ],
    )


def _make_deg_call():
    mesh = plsc.VectorSubcoreMesh(core_axis_name="c", subcore_axis_name="s")
    return pl.kernel(
        _deg_kernel,
        out_type=jax.ShapeDtypeStruct((2 * NR, 16), jnp.float32),
        mesh=mesh,
        scratch_types=[
            pltpu.VMEM((E_PAD // (32 * CHUNK), CHUNK), jnp.int32),
            pltpu.VMEM((CHUNK, 16), jnp.float32),
            pltpu.VMEM((CHUNK, 16), jnp.float32),
            pltpu.VMEM_SHARED((NR, 16), jnp.float32),
        ],
    )


def _tc1_body(x_ref, w_ref, degp_ref, hw_ref, dinv_ref):
    deg = degp_ref[0:NR, 0:1] + degp_ref[NR:2 * NR, 0:1] + 1.0
    dinv = lax.rsqrt(deg)
    dinv_ref[...] = dinv
    hw_ref[...] = jnp.dot(x_ref[...], w_ref[...],
                          preferred_element_type=jnp.float32) * dinv


def _tc_u2_body(sp_ref, hw_ref, dinv_ref, b_ref, u2_ref):
    dinv = dinv_ref[...]
    ssum = sp_ref[0:NR, :] + sp_ref[NR:2 * NR, :]
    h = jnp.maximum(dinv * (ssum + hw_ref[...]) + b_ref[0:1, :], 0.0)
    u2_ref[...] = h * dinv


def _tc_u3_body(sp_ref, u2_ref, dinv_ref, b_ref, w_ref, u3_ref):
    dinv = dinv_ref[...]
    ssum = sp_ref[0:NR, :] + sp_ref[NR:2 * NR, :] + u2_ref[...]
    t = jnp.dot(ssum, w_ref[...], preferred_element_type=jnp.float32)
    u3_ref[...] = jnp.maximum(dinv * t + b_ref[0:1, :], 0.0) * dinv


def _tc_h3_body(sp_ref, u3_ref, dinv_ref, b_ref, w_ref, h3_ref):
    dinv = dinv_ref[...]
    ssum = sp_ref[0:NR, :] + sp_ref[NR:2 * NR, :] + u3_ref[...]
    t = jnp.dot(ssum, w_ref[...], preferred_element_type=jnp.float32)
    h3_ref[...] = jnp.maximum(dinv * t + b_ref[0:1, :], 0.0)


def _tc_gate_body(h3_ref, gw1_ref, gb1_ref, gw2_ref, gb2_ref, gw3_ref,
                  gb3_ref, gate_ref):
    g = jnp.maximum(jnp.dot(h3_ref[...], gw1_ref[...],
                            preferred_element_type=jnp.float32)
                    + gb1_ref[0:1, :], 0.0)
    g = jnp.maximum(jnp.dot(g, gw2_ref[...],
                            preferred_element_type=jnp.float32)
                    + gb2_ref[0:1, :], 0.0)
    gate_ref[...] = jnp.dot(g, gw3_ref[...],
                            preferred_element_type=jnp.float32) + gb3_ref[0:1, :]


def _tc_pool_body(h3_ref, gate_ref, batch_ref, hw1_ref, hb1_ref, hw2_ref,
                  hb2_ref, hw3_ref, hb3_ref, o_ref):
    batch = batch_ref[...]
    seg = lax.broadcasted_iota(jnp.int32, (NR, G), 1)
    m = (batch == seg).astype(jnp.float32)
    valid = (batch < G)
    gate = gate_ref[...]
    gb = jnp.where(m > 0.0, gate, -1e30)
    smax = jnp.max(gb, axis=0, keepdims=True)
    smax_node = jnp.sum(m * smax, axis=1, keepdims=True)
    e = jnp.where(valid, jnp.exp(gate - smax_node), 0.0)
    den = jnp.sum(m * e, axis=0, keepdims=True)
    den_node = jnp.sum(m * den, axis=1, keepdims=True)
    alpha = e / jnp.where(den_node > 0.0, den_node, 1.0)
    ah = alpha * h3_ref[...]
    pooled = lax.dot_general(m, ah, (((0,), (0,)), ((), ())),
                             preferred_element_type=jnp.float32)
    o = jnp.maximum(jnp.dot(pooled, hw1_ref[...],
                            preferred_element_type=jnp.float32)
                    + hb1_ref[0:1, :], 0.0)
    o = jnp.maximum(jnp.dot(o, hw2_ref[...],
                            preferred_element_type=jnp.float32)
                    + hb2_ref[0:1, :], 0.0)
    o_ref[...] = jnp.dot(o, hw3_ref[...],
                         preferred_element_type=jnp.float32) + hb3_ref[0:1, :]


def _tc_call(body, out_shapes):
    return pl.pallas_call(body, out_shape=out_shapes)


def kernel(x, edge_index, batch, W1, b1, W2, b2, W3, b3, gW1, gb1, gW2, gb2,
           gW3, gb3, hW1, hb1, hW2, hb2, hW3, hb3):
    x_pad = jnp.zeros((NR, 128), jnp.float32).at[:N, :].set(x)
    src2d = jnp.full((E_PAD,), N, jnp.int32).at[:E].set(edge_index[0])
    src2d = src2d.reshape(E_PAD // CK, CK)
    dst2d = jnp.full((E_PAD,), N, jnp.int32).at[:E].set(edge_index[1])
    dst2d = dst2d.reshape(E_PAD // CK, CK)
    dst2d_deg = dst2d.reshape(E_PAD // CHUNK, CHUNK)
    batch_pad = jnp.full((NR,), G, jnp.int32).at[:N].set(batch)
    batch_pad = batch_pad.reshape(NR, 1)
    b1r = b1.reshape(1, -1)
    b2r = b2.reshape(1, -1)
    b3r = b3.reshape(1, -1)
    gb1r, gb2r, gb3r = gb1.reshape(1, -1), gb2.reshape(1, -1), gb3.reshape(1, -1)
    hb1r, hb2r, hb3r = hb1.reshape(1, -1), hb2.reshape(1, -1), hb3.reshape(1, -1)

    degp = _make_deg_call()(dst2d_deg)

    hw1, dinv = _tc_call(
        _tc1_body,
        [jax.ShapeDtypeStruct((NR, 64), jnp.float32),
         jax.ShapeDtypeStruct((NR, 1), jnp.float32)])(x_pad, W1, degp)

    sp1 = _make_layer_es2_call(64)(hw1, src2d, dst2d)
    u2 = _tc_call(
        _tc_u2_body,
        jax.ShapeDtypeStruct((NR, 64), jnp.float32))(sp1, hw1, dinv, b1r)

    sp2 = _make_layer_es2_call(64)(u2, src2d, dst2d)
    u3 = _tc_call(
        _tc_u3_body,
        jax.ShapeDtypeStruct((NR, 128), jnp.float32))(
            sp2, u2, dinv, b2r, W2)

    sp3 = _make_layer_es_call(128)(u3, src2d, dst2d)
    h3 = _tc_call(
        _tc_h3_body,
        jax.ShapeDtypeStruct((NR, 256), jnp.float32))(
            sp3, u3, dinv, b3r, W3)

    gate = _tc_call(
        _tc_gate_body,
        jax.ShapeDtypeStruct((NR, 1), jnp.float32))(
            h3, gW1, gb1r, gW2, gb2r, gW3, gb3r)
    o = _tc_call(
        _tc_pool_body,
        jax.ShapeDtypeStruct((G, 1), jnp.float32))(
            h3, gate, batch_pad, hW1, hb1r, hW2, hb2r, hW3, hb3r)
    return o.reshape(-1)
